# baseline (speedup 1.0000x reference)
import functools

import jax
import jax.numpy as jnp
from jax import lax
from jax.experimental import pallas as pl
from jax.experimental.pallas import tpu as pltpu
from jax.experimental.pallas import tpu_sc as plsc

NC = 2
NS = 16
NW = NC * NS
SEQ_PAD = 64
SEQ_CHUNK = 16


def _gather_body(n_seq, seq_len, emb, x_hbm, table_hbm, out_hbm,
                 idx_v, rows_v, sem):
    wid = lax.axis_index("s") * NC + lax.axis_index("c")
    seq_per_w = n_seq // NW
    n_chunks = seq_per_w // SEQ_CHUNK
    seq0 = wid * seq_per_w
    rows_per_chunk = SEQ_CHUNK * SEQ_PAD // 128
    row0 = seq0 * SEQ_PAD // 128

    gl = (seq_len + 7) // 8 * 8

    def chunk(i, carry):
        s = seq0 + i * SEQ_CHUNK
        pltpu.sync_copy(x_hbm.at[pl.ds(row0 + i * rows_per_chunk,
                                       rows_per_chunk)], idx_v)
        descs = [
            pltpu.async_copy(
                table_hbm.at[idx_v.at[q // 2, pl.ds((q % 2) * SEQ_PAD, gl)]],
                rows_v.at[q],
                sem,
            )
            for q in range(SEQ_CHUNK)
        ]
        for d in descs:
            d.wait()
        pltpu.sync_copy(rows_v.at[slice(None), pl.ds(0, seq_len)],
                        out_hbm.at[pl.ds(s, SEQ_CHUNK)])
        return carry

    lax.fori_loop(0, n_chunks, chunk, 0)


def kernel(x, table):
    n_seq, seq_len = x.shape
    vocab, emb = table.shape

    xp = jnp.pad(x & 0x7FFFFFFF, ((0, 0), (0, SEQ_PAD - seq_len)))
    xp = xp.reshape(n_seq * SEQ_PAD // 128, 128)

    embed = pl.kernel(
        functools.partial(_gather_body, n_seq, seq_len, emb),
        out_type=jax.ShapeDtypeStruct((n_seq, seq_len, emb), jnp.float32),
        mesh=plsc.VectorSubcoreMesh(core_axis_name="c", subcore_axis_name="s"),
        compiler_params=pltpu.CompilerParams(use_tc_tiling_on_sc=False),
        scratch_types=[
            pltpu.VMEM((SEQ_CHUNK * SEQ_PAD // 128, 128), jnp.int32),
            pltpu.VMEM((SEQ_CHUNK, (seq_len + 7) // 8 * 8, emb), jnp.float32),
            pltpu.SemaphoreType.DMA,
        ],
    )
    return embed(xp, table)
